# baseline (speedup 1.0000x reference)
import functools

import jax
import jax.numpy as jnp
from jax.experimental import pallas as pl

_B = 2
_NPER = 4096
_BIG = 1e9


def _fps_body(xyzt_ref, xyzr_ref, out_ref, *, npoint, n, b):
    xt = xyzt_ref[...]
    xr = xyzr_ref[...]
    out_ref[:, 0:1, :] = xr[:, 0:1, :]
    iota = jax.lax.broadcasted_iota(jnp.int32, (b, n), 1)
    col0 = jnp.sum(
        xt * (iota == 0).astype(jnp.float32)[:, None, :], axis=2, keepdims=True
    )

    def body(i, carry):
        dists, lastcol = carry
        d = jnp.sum((xt - lastcol) ** 2, axis=1)
        dists = jnp.minimum(dists, d)
        mx = jnp.max(dists, axis=1, keepdims=True)
        nxt = jnp.min(
            jnp.where(dists == mx, iota, jnp.int32(n)), axis=1, keepdims=True
        )
        oh = (iota == nxt).astype(jnp.float32)
        nxtcol = jnp.sum(xt * oh[:, None, :], axis=2, keepdims=True)
        row = jax.lax.dot_general(
            oh[:, None, :], xr, (((2,), (1,)), ((0,), (0,))),
            precision=jax.lax.Precision.HIGHEST,
            preferred_element_type=jnp.float32,
        )
        out_ref[:, pl.ds(i, 1), :] = row
        return dists, nxtcol

    dists0 = jnp.full((b, n), 1e10, dtype=jnp.float32)
    jax.lax.fori_loop(1, npoint, body, (dists0, col0))


def _fps(xyz_rows, npoint):
    b, n, _ = xyz_rows.shape
    xyzt = jnp.transpose(xyz_rows, (0, 2, 1))
    return pl.pallas_call(
        functools.partial(_fps_body, npoint=npoint, n=n, b=b),
        grid=(1,),
        in_specs=[
            pl.BlockSpec((b, 8, n), lambda i: (0, 0, 0)),
            pl.BlockSpec((b, n, 8), lambda i: (0, 0, 0)),
        ],
        out_specs=pl.BlockSpec((b, npoint, 8), lambda i: (0, 0, 0)),
        out_shape=jax.ShapeDtypeStruct((b, npoint, 8), jnp.float32),
    )(xyzt, xyz_rows)


def _group_body(new_ref, tabt_ref, out_ref, *, r2, k, n, ct, mb):
    new = new_ref[0]
    tabt = tabt_ref[0]
    d2 = (new[:, 0:1] - tabt[0:1, :]) ** 2
    d2 = d2 + (new[:, 1:2] - tabt[1:2, :]) ** 2
    d2 = d2 + (new[:, 2:3] - tabt[2:3, :]) ** 2
    big_i = jnp.int32(1 << 30)
    iota = jax.lax.broadcasted_iota(jnp.int32, (mb, n), 1)
    score = jnp.where(d2 < r2, iota, big_i)
    m0 = jnp.min(score, axis=1, keepdims=True)
    valid0 = m0 < big_i
    sel0 = (score == m0) & valid0
    oh0 = sel0.astype(jnp.float32)
    offs = jnp.where(valid0, new[:, 0:3], 0.0)
    offs_full = jnp.concatenate(
        [offs, jnp.zeros((mb, ct - 3), jnp.float32)], axis=1
    )
    cur = score
    for s in range(k):
        if s == 0:
            selb = sel0
            oh = oh0
        else:
            ms = jnp.min(cur, axis=1, keepdims=True)
            vs = ms < big_i
            selb = (cur == ms) & vs
            oh = jnp.where(vs, selb.astype(jnp.float32), oh0)
        g = jax.lax.dot_general(
            oh, tabt, (((1,), (1,)), ((), ())),
            precision=jax.lax.Precision.HIGHEST,
            preferred_element_type=jnp.float32,
        )
        out_ref[0, s, :, :] = g - offs_full
        cur = jnp.where(selb, big_i, cur)


def _group(new_rows, tabt, radius, k, mb):
    b, m, _ = new_rows.shape
    ct, n = tabt.shape[1], tabt.shape[2]
    return pl.pallas_call(
        functools.partial(
            _group_body, r2=radius * radius, k=k, n=n, ct=ct, mb=mb
        ),
        grid=(b, m // mb),
        in_specs=[
            pl.BlockSpec((1, mb, 8), lambda i, j: (i, j, 0)),
            pl.BlockSpec((1, ct, n), lambda i, j: (i, 0, 0)),
        ],
        out_specs=pl.BlockSpec((1, k, mb, ct), lambda i, j: (i, 0, j, 0)),
        out_shape=jax.ShapeDtypeStruct((b, k, m, ct), jnp.float32),
    )(new_rows, tabt)


def _layer_body(x_ref, st_ref, w_ref, y_ref, sum_ref, sq_ref, *, act):
    x = x_ref[...]
    a = x * st_ref[0:1, :] + st_ref[1:2, :]
    if act:
        a = jnp.maximum(a, 0.0)
    y = jax.lax.dot_general(
        a, w_ref[...], (((1,), (0,)), ((), ())), preferred_element_type=jnp.float32
    )
    y_ref[...] = y

    @pl.when(pl.program_id(0) == 0)
    def _():
        sum_ref[...] = jnp.zeros_like(sum_ref)
        sq_ref[...] = jnp.zeros_like(sq_ref)

    sum_ref[...] += jnp.sum(y, axis=0, keepdims=True)
    sq_ref[...] += jnp.sum(y * y, axis=0, keepdims=True)


def _layer(x, st, wt, act, rb):
    r, ci = x.shape
    co = wt.shape[1]
    rb = min(rb, r)
    return pl.pallas_call(
        functools.partial(_layer_body, act=act),
        grid=(r // rb,),
        in_specs=[
            pl.BlockSpec((rb, ci), lambda i: (i, 0)),
            pl.BlockSpec((8, ci), lambda i: (0, 0)),
            pl.BlockSpec((ci, co), lambda i: (0, 0)),
        ],
        out_specs=[
            pl.BlockSpec((rb, co), lambda i: (i, 0)),
            pl.BlockSpec((1, co), lambda i: (0, 0)),
            pl.BlockSpec((1, co), lambda i: (0, 0)),
        ],
        out_shape=[
            jax.ShapeDtypeStruct((r, co), jnp.float32),
            jax.ShapeDtypeStruct((1, co), jnp.float32),
            jax.ShapeDtypeStruct((1, co), jnp.float32),
        ],
    )(x, st, wt)


def _pack_st(s, t):
    st = jnp.zeros((8, s.shape[-1]), jnp.float32)
    return st.at[0].set(s).at[1].set(t)


def _bn_affine(ssum, ssq, ntot, g, b):
    mean = ssum[0] / ntot
    var = jnp.maximum(ssq[0] / ntot - mean * mean, 0.0)
    s = g * jax.lax.rsqrt(var + 1e-5)
    t = b - mean * s
    return _pack_st(s, t)


def _mlp_chain(x, layers, rb=2048):
    r = x.shape[0]
    st = _pack_st(jnp.ones((x.shape[1],), jnp.float32), jnp.zeros((x.shape[1],), jnp.float32))
    act = False
    y = x
    for lyr in layers:
        y, ssum, ssq = _layer(y, st, lyr["W"].T, act, rb)
        st = _bn_affine(ssum, ssq, float(r), lyr["g"], lyr["b"])
        act = True
    return y, st


def _finmax_body(y_ref, st_ref, o_ref, *, mb, k):
    a = jnp.maximum(y_ref[...] * st_ref[0:1, :] + st_ref[1:2, :], 0.0)
    c = a.shape[-1]
    o_ref[...] = jnp.max(a.reshape(mb, k, c), axis=1)


def _finalize_max(y, st, k):
    rk, c = y.shape
    rows = rk // k
    mb = min(rows, 256)
    return pl.pallas_call(
        functools.partial(_finmax_body, mb=mb, k=k),
        grid=(rows // mb,),
        in_specs=[
            pl.BlockSpec((mb * k, c), lambda i: (i, 0)),
            pl.BlockSpec((8, c), lambda i: (0, 0)),
        ],
        out_specs=pl.BlockSpec((mb, c), lambda i: (i, 0)),
        out_shape=jax.ShapeDtypeStruct((rows, c), jnp.float32),
    )(y, st)


def _finmax2_body(y_ref, st_ref, o_ref):
    a = jnp.maximum(y_ref[0] * st_ref[0:1, :] + st_ref[1:2, :], 0.0)
    o_ref[0] = jnp.max(a, axis=0)


def _finalize_max2(y4, st, mbf=128):
    b, k, m, c = y4.shape
    mbf = min(mbf, m)
    return pl.pallas_call(
        _finmax2_body,
        grid=(b, m // mbf),
        in_specs=[
            pl.BlockSpec((1, k, mbf, c), lambda i, j: (i, 0, j, 0)),
            pl.BlockSpec((8, c), lambda i, j: (0, 0)),
        ],
        out_specs=pl.BlockSpec((1, mbf, c), lambda i, j: (i, j, 0)),
        out_shape=jax.ShapeDtypeStruct((b, m, c), jnp.float32),
    )(y4, st)


def _fin_body(y_ref, st_ref, o_ref):
    o_ref[...] = jnp.maximum(y_ref[...] * st_ref[0:1, :] + st_ref[1:2, :], 0.0)


def _finalize(y, st, rb=2048):
    r, c = y.shape
    rb = min(rb, r)
    return pl.pallas_call(
        _fin_body,
        grid=(r // rb,),
        in_specs=[
            pl.BlockSpec((rb, c), lambda i: (i, 0)),
            pl.BlockSpec((8, c), lambda i: (0, 0)),
        ],
        out_specs=pl.BlockSpec((rb, c), lambda i: (i, 0)),
        out_shape=jax.ShapeDtypeStruct((r, c), jnp.float32),
    )(y, st)


def _sa_msg(new_rows, tabt, radii, nsamps, branches, mb):
    b, m, _ = new_rows.shape
    ct = tabt.shape[1]
    outs = []
    for radius, k, layers in zip(radii, nsamps, branches):
        g = _group(new_rows, tabt, radius, k, mb)
        x = g.reshape(b * k * m, ct)
        y, st = _mlp_chain(x, layers)
        c = y.shape[-1]
        outs.append(_finalize_max2(y.reshape(b, k, m, c), st))
    return jnp.concatenate(outs, axis=-1)


def _interp_body(unk_ref, kt_ref, kf_ref, skip_ref, out_ref, *, kn, c, ub):
    unk = unk_ref[0]
    kt = kt_ref[0]
    d2 = (unk[:, 0:1] - kt[0:1, :]) ** 2
    d2 = d2 + (unk[:, 1:2] - kt[1:2, :]) ** 2
    d2 = d2 + (unk[:, 2:3] - kt[2:3, :]) ** 2
    iota = jax.lax.broadcasted_iota(jnp.int32, (ub, kn), 1)
    wmat = jnp.zeros((ub, kn), jnp.float32)
    wsum = jnp.zeros((ub, 1), jnp.float32)
    cur = d2
    for _ in range(3):
        ms = jnp.min(cur, axis=1, keepdims=True)
        idx = jnp.min(
            jnp.where(cur == ms, iota, jnp.int32(kn)), axis=1, keepdims=True
        )
        oh = iota == idx
        w = 1.0 / (ms + 1e-8)
        wmat = wmat + jnp.where(oh, w, 0.0)
        wsum = wsum + w
        cur = jnp.where(oh, _BIG, cur)
    wmat = wmat / wsum
    interp = jax.lax.dot_general(
        wmat, kf_ref[0], (((1,), (0,)), ((), ())),
        precision=jax.lax.Precision.HIGHEST,
        preferred_element_type=jnp.float32,
    )
    out_ref[0, :, 0:c] = interp
    out_ref[0, :, c:] = skip_ref[0]


def _interp_concat(unk_rows, known_rows, kf, skip, ub):
    b, u, _ = unk_rows.shape
    kn = known_rows.shape[1]
    c = kf.shape[2]
    cu = skip.shape[2]
    kt = jnp.transpose(known_rows, (0, 2, 1))
    return pl.pallas_call(
        functools.partial(_interp_body, kn=kn, c=c, ub=ub),
        grid=(b, u // ub),
        in_specs=[
            pl.BlockSpec((1, ub, 8), lambda i, j: (i, j, 0)),
            pl.BlockSpec((1, 8, kn), lambda i, j: (i, 0, 0)),
            pl.BlockSpec((1, kn, c), lambda i, j: (i, 0, 0)),
            pl.BlockSpec((1, ub, cu), lambda i, j: (i, j, 0)),
        ],
        out_specs=pl.BlockSpec((1, ub, c + cu), lambda i, j: (i, j, 0)),
        out_shape=jax.ShapeDtypeStruct((b, u, c + cu), jnp.float32),
    )(unk_rows, kt, kf, skip)


def _fp(unk_rows, known_rows, kf, skip, layers, ub):
    x = _interp_concat(unk_rows, known_rows, kf, skip, ub)
    b, u, cx = x.shape
    y, st = _mlp_chain(x.reshape(b * u, cx), layers)
    return _finalize(y, st)


def kernel(points, params):
    points = jnp.asarray(points, jnp.float32)
    xyz = points[:, 1:4]
    xb = xyz.reshape(_B, _NPER, 3)
    xb_rows = jnp.pad(xb, ((0, 0), (0, 0), (0, 5)))
    feats = points[:, 4:].reshape(_B, _NPER, -1)

    nx0_rows = _fps(xb_rows, 1024)
    tabt0 = jnp.concatenate(
        [jnp.transpose(xb, (0, 2, 1)), jnp.transpose(feats, (0, 2, 1))], axis=1
    )
    f0b = _sa_msg(
        nx0_rows, tabt0, [0.4, 0.8], [16, 32], params["sa"][0], mb=64
    )

    nx1_rows = _fps(nx0_rows, 256)
    tabt1 = jnp.concatenate(
        [jnp.transpose(nx0_rows[:, :, :3], (0, 2, 1)), jnp.transpose(f0b, (0, 2, 1))],
        axis=1,
    )
    f1b = _sa_msg(
        nx1_rows, tabt1, [0.8, 1.6], [16, 32], params["sa"][1], mb=128
    )

    f0u = _fp(nx0_rows, nx1_rows, f1b, f0b, params["fp"][1], ub=256)
    raw = points[:, 1:].reshape(_B, _NPER, -1)
    pf = _fp(
        xb_rows, nx0_rows, f0u.reshape(_B, 1024, 128), raw, params["fp"][0], ub=256
    )

    gq_rows = jnp.repeat(nx1_rows[:, 0:1, :], 8, axis=1)
    tabtg = jnp.concatenate(
        [jnp.transpose(nx1_rows[:, :, :3], (0, 2, 1)), jnp.transpose(f1b, (0, 2, 1))],
        axis=1,
    )
    gg = _group(gq_rows, tabtg, 100.0, 64, mb=8)
    xg = gg[:, :, 0, :].reshape(_B * 64, 259)
    yg, stg = _mlp_chain(xg, params["gsa"])
    gf = _finalize_max(yg, stg, 64)

    point_coords = points[:, 0:4]
    return pf, point_coords, gf


# speedup vs baseline: 1.6196x; 1.6196x over previous
import functools

import jax
import jax.numpy as jnp
from jax.experimental import pallas as pl

_B = 2
_NPER = 4096
_BIG = 1e9


def _fps_body(xyzt_ref, xyzr_ref, out_ref, *, npoint, n, b):
    xt = xyzt_ref[...]
    out_ref[:, 0:1, :] = xyzr_ref[:, 0:1, :]
    iota = jax.lax.broadcasted_iota(jnp.int32, (b, n), 1)
    col0 = jnp.sum(
        xt * (iota == 0).astype(jnp.float32)[:, None, :], axis=2, keepdims=True
    )

    def body(i, carry):
        dists, lastcol = carry
        d = jnp.sum((xt - lastcol) ** 2, axis=1)
        dists = jnp.minimum(dists, d)
        mx = jnp.max(dists, axis=1, keepdims=True)
        nxt = jnp.min(
            jnp.where(dists == mx, iota, jnp.int32(n)), axis=1, keepdims=True
        )
        oh = (iota == nxt).astype(jnp.float32)
        nxtcol = jnp.sum(xt * oh[:, None, :], axis=2, keepdims=True)
        out_ref[:, pl.ds(i, 1), :] = jnp.transpose(nxtcol, (0, 2, 1))
        return dists, nxtcol

    dists0 = jnp.full((b, n), 1e10, dtype=jnp.float32)
    jax.lax.fori_loop(1, npoint, body, (dists0, col0))


def _fps(xyz_rows, npoint):
    b, n, _ = xyz_rows.shape
    xyzt = jnp.transpose(xyz_rows, (0, 2, 1))
    return pl.pallas_call(
        functools.partial(_fps_body, npoint=npoint, n=n, b=b),
        grid=(1,),
        in_specs=[
            pl.BlockSpec((b, 8, n), lambda i: (0, 0, 0)),
            pl.BlockSpec((b, n, 8), lambda i: (0, 0, 0)),
        ],
        out_specs=pl.BlockSpec((b, npoint, 8), lambda i: (0, 0, 0)),
        out_shape=jax.ShapeDtypeStruct((b, npoint, 8), jnp.float32),
    )(xyzt, xyz_rows)


def _group_body(new_ref, tabt_ref, out_ref, *, r2, k, n, ct, mb):
    new = new_ref[0]
    tabt = tabt_ref[0]
    d2 = (new[:, 0:1] - tabt[0:1, :]) ** 2
    d2 = d2 + (new[:, 1:2] - tabt[1:2, :]) ** 2
    d2 = d2 + (new[:, 2:3] - tabt[2:3, :]) ** 2
    big_i = jnp.int32(1 << 30)
    iota = jax.lax.broadcasted_iota(jnp.int32, (mb, n), 1)
    score = jnp.where(d2 < r2, iota, big_i)
    m0 = jnp.min(score, axis=1, keepdims=True)
    valid0 = m0 < big_i
    sel0 = (score == m0) & valid0
    oh0 = sel0.astype(jnp.float32)
    offs = jnp.where(valid0, new[:, 0:3], 0.0)
    offs_full = jnp.concatenate(
        [offs, jnp.zeros((mb, ct - 3), jnp.float32)], axis=1
    )
    cur = score
    for s in range(k):
        if s == 0:
            selb = sel0
            oh = oh0
        else:
            ms = jnp.min(cur, axis=1, keepdims=True)
            vs = ms < big_i
            selb = (cur == ms) & vs
            oh = jnp.where(vs, selb.astype(jnp.float32), oh0)
        g = jax.lax.dot_general(
            oh, tabt, (((1,), (1,)), ((), ())),
            precision=jax.lax.Precision.HIGHEST,
            preferred_element_type=jnp.float32,
        )
        out_ref[0, s, :, :] = g - offs_full
        cur = jnp.where(selb, big_i, cur)


def _group(new_rows, tabt, radius, k, mb):
    b, m, _ = new_rows.shape
    ct, n = tabt.shape[1], tabt.shape[2]
    return pl.pallas_call(
        functools.partial(
            _group_body, r2=radius * radius, k=k, n=n, ct=ct, mb=mb
        ),
        grid=(b, m // mb),
        in_specs=[
            pl.BlockSpec((1, mb, 8), lambda i, j: (i, j, 0)),
            pl.BlockSpec((1, ct, n), lambda i, j: (i, 0, 0)),
        ],
        out_specs=pl.BlockSpec((1, k, mb, ct), lambda i, j: (i, 0, j, 0)),
        out_shape=jax.ShapeDtypeStruct((b, k, m, ct), jnp.float32),
    )(new_rows, tabt)


def _layer_body(x_ref, st_ref, w_ref, y_ref, sum_ref, sq_ref, *, act):
    x = x_ref[...]
    a = x * st_ref[0:1, :] + st_ref[1:2, :]
    if act:
        a = jnp.maximum(a, 0.0)
    y = jax.lax.dot_general(
        a, w_ref[...], (((1,), (0,)), ((), ())), preferred_element_type=jnp.float32
    )
    y_ref[...] = y

    @pl.when(pl.program_id(0) == 0)
    def _():
        sum_ref[...] = jnp.zeros_like(sum_ref)
        sq_ref[...] = jnp.zeros_like(sq_ref)

    sum_ref[...] += jnp.sum(y, axis=0, keepdims=True)
    sq_ref[...] += jnp.sum(y * y, axis=0, keepdims=True)


def _layer(x, st, wt, act, rb):
    r, ci = x.shape
    co = wt.shape[1]
    rb = min(rb, r)
    return pl.pallas_call(
        functools.partial(_layer_body, act=act),
        grid=(r // rb,),
        in_specs=[
            pl.BlockSpec((rb, ci), lambda i: (i, 0)),
            pl.BlockSpec((8, ci), lambda i: (0, 0)),
            pl.BlockSpec((ci, co), lambda i: (0, 0)),
        ],
        out_specs=[
            pl.BlockSpec((rb, co), lambda i: (i, 0)),
            pl.BlockSpec((1, co), lambda i: (0, 0)),
            pl.BlockSpec((1, co), lambda i: (0, 0)),
        ],
        out_shape=[
            jax.ShapeDtypeStruct((r, co), jnp.float32),
            jax.ShapeDtypeStruct((1, co), jnp.float32),
            jax.ShapeDtypeStruct((1, co), jnp.float32),
        ],
    )(x, st, wt)


def _pack_st(s, t):
    st = jnp.zeros((8, s.shape[-1]), jnp.float32)
    return st.at[0].set(s).at[1].set(t)


def _bn_affine(ssum, ssq, ntot, g, b):
    mean = ssum[0] / ntot
    var = jnp.maximum(ssq[0] / ntot - mean * mean, 0.0)
    s = g * jax.lax.rsqrt(var + 1e-5)
    t = b - mean * s
    return _pack_st(s, t)


def _mlp_chain(x, layers, rb=2048):
    r = x.shape[0]
    st = _pack_st(jnp.ones((x.shape[1],), jnp.float32), jnp.zeros((x.shape[1],), jnp.float32))
    act = False
    y = x
    for lyr in layers:
        y, ssum, ssq = _layer(y, st, lyr["W"].T, act, rb)
        st = _bn_affine(ssum, ssq, float(r), lyr["g"], lyr["b"])
        act = True
    return y, st


def _finmax_body(y_ref, st_ref, o_ref, *, mb, k):
    a = jnp.maximum(y_ref[...] * st_ref[0:1, :] + st_ref[1:2, :], 0.0)
    c = a.shape[-1]
    o_ref[...] = jnp.max(a.reshape(mb, k, c), axis=1)


def _finalize_max(y, st, k):
    rk, c = y.shape
    rows = rk // k
    mb = min(rows, 256)
    return pl.pallas_call(
        functools.partial(_finmax_body, mb=mb, k=k),
        grid=(rows // mb,),
        in_specs=[
            pl.BlockSpec((mb * k, c), lambda i: (i, 0)),
            pl.BlockSpec((8, c), lambda i: (0, 0)),
        ],
        out_specs=pl.BlockSpec((mb, c), lambda i: (i, 0)),
        out_shape=jax.ShapeDtypeStruct((rows, c), jnp.float32),
    )(y, st)


def _finmax2_body(y_ref, st_ref, o_ref):
    a = jnp.maximum(y_ref[0] * st_ref[0:1, :] + st_ref[1:2, :], 0.0)
    o_ref[0] = jnp.max(a, axis=0)


def _finalize_max2(y4, st, mbf=128):
    b, k, m, c = y4.shape
    mbf = min(mbf, m)
    return pl.pallas_call(
        _finmax2_body,
        grid=(b, m // mbf),
        in_specs=[
            pl.BlockSpec((1, k, mbf, c), lambda i, j: (i, 0, j, 0)),
            pl.BlockSpec((8, c), lambda i, j: (0, 0)),
        ],
        out_specs=pl.BlockSpec((1, mbf, c), lambda i, j: (i, j, 0)),
        out_shape=jax.ShapeDtypeStruct((b, m, c), jnp.float32),
    )(y4, st)


def _fin_body(y_ref, st_ref, o_ref):
    o_ref[...] = jnp.maximum(y_ref[...] * st_ref[0:1, :] + st_ref[1:2, :], 0.0)


def _finalize(y, st, rb=2048):
    r, c = y.shape
    rb = min(rb, r)
    return pl.pallas_call(
        _fin_body,
        grid=(r // rb,),
        in_specs=[
            pl.BlockSpec((rb, c), lambda i: (i, 0)),
            pl.BlockSpec((8, c), lambda i: (0, 0)),
        ],
        out_specs=pl.BlockSpec((rb, c), lambda i: (i, 0)),
        out_shape=jax.ShapeDtypeStruct((r, c), jnp.float32),
    )(y, st)


def _sa_msg(new_rows, tabt, radii, nsamps, branches, mb):
    b, m, _ = new_rows.shape
    ct = tabt.shape[1]
    outs = []
    for radius, k, layers in zip(radii, nsamps, branches):
        g = _group(new_rows, tabt, radius, k, mb)
        x = g.reshape(b * k * m, ct)
        y, st = _mlp_chain(x, layers)
        c = y.shape[-1]
        outs.append(_finalize_max2(y.reshape(b, k, m, c), st))
    return jnp.concatenate(outs, axis=-1)


def _interp_body(unk_ref, kt_ref, kf_ref, skip_ref, out_ref, *, kn, c, ub):
    unk = unk_ref[0]
    kt = kt_ref[0]
    d2 = (unk[:, 0:1] - kt[0:1, :]) ** 2
    d2 = d2 + (unk[:, 1:2] - kt[1:2, :]) ** 2
    d2 = d2 + (unk[:, 2:3] - kt[2:3, :]) ** 2
    iota = jax.lax.broadcasted_iota(jnp.int32, (ub, kn), 1)
    wmat = jnp.zeros((ub, kn), jnp.float32)
    wsum = jnp.zeros((ub, 1), jnp.float32)
    cur = d2
    for _ in range(3):
        ms = jnp.min(cur, axis=1, keepdims=True)
        idx = jnp.min(
            jnp.where(cur == ms, iota, jnp.int32(kn)), axis=1, keepdims=True
        )
        oh = iota == idx
        w = 1.0 / (ms + 1e-8)
        wmat = wmat + jnp.where(oh, w, 0.0)
        wsum = wsum + w
        cur = jnp.where(oh, _BIG, cur)
    wmat = wmat / wsum
    interp = jax.lax.dot_general(
        wmat, kf_ref[0], (((1,), (0,)), ((), ())),
        precision=jax.lax.Precision.HIGHEST,
        preferred_element_type=jnp.float32,
    )
    out_ref[0, :, 0:c] = interp
    out_ref[0, :, c:] = skip_ref[0]


def _interp_concat(unk_rows, known_rows, kf, skip, ub):
    b, u, _ = unk_rows.shape
    kn = known_rows.shape[1]
    c = kf.shape[2]
    cu = skip.shape[2]
    kt = jnp.transpose(known_rows, (0, 2, 1))
    return pl.pallas_call(
        functools.partial(_interp_body, kn=kn, c=c, ub=ub),
        grid=(b, u // ub),
        in_specs=[
            pl.BlockSpec((1, ub, 8), lambda i, j: (i, j, 0)),
            pl.BlockSpec((1, 8, kn), lambda i, j: (i, 0, 0)),
            pl.BlockSpec((1, kn, c), lambda i, j: (i, 0, 0)),
            pl.BlockSpec((1, ub, cu), lambda i, j: (i, j, 0)),
        ],
        out_specs=pl.BlockSpec((1, ub, c + cu), lambda i, j: (i, j, 0)),
        out_shape=jax.ShapeDtypeStruct((b, u, c + cu), jnp.float32),
    )(unk_rows, kt, kf, skip)


def _fp(unk_rows, known_rows, kf, skip, layers, ub):
    x = _interp_concat(unk_rows, known_rows, kf, skip, ub)
    b, u, cx = x.shape
    y, st = _mlp_chain(x.reshape(b * u, cx), layers)
    return _finalize(y, st)


def kernel(points, params):
    points = jnp.asarray(points, jnp.float32)
    xyz = points[:, 1:4]
    xb = xyz.reshape(_B, _NPER, 3)
    xb_rows = jnp.pad(xb, ((0, 0), (0, 0), (0, 5)))
    feats = points[:, 4:].reshape(_B, _NPER, -1)

    nx0_rows = _fps(xb_rows, 1024)
    tabt0 = jnp.concatenate(
        [jnp.transpose(xb, (0, 2, 1)), jnp.transpose(feats, (0, 2, 1))], axis=1
    )
    f0b = _sa_msg(
        nx0_rows, tabt0, [0.4, 0.8], [16, 32], params["sa"][0], mb=64
    )

    nx1_rows = _fps(nx0_rows, 256)
    tabt1 = jnp.concatenate(
        [jnp.transpose(nx0_rows[:, :, :3], (0, 2, 1)), jnp.transpose(f0b, (0, 2, 1))],
        axis=1,
    )
    f1b = _sa_msg(
        nx1_rows, tabt1, [0.8, 1.6], [16, 32], params["sa"][1], mb=128
    )

    f0u = _fp(nx0_rows, nx1_rows, f1b, f0b, params["fp"][1], ub=256)
    raw = points[:, 1:].reshape(_B, _NPER, -1)
    pf = _fp(
        xb_rows, nx0_rows, f0u.reshape(_B, 1024, 128), raw, params["fp"][0], ub=256
    )

    gq_rows = jnp.repeat(nx1_rows[:, 0:1, :], 8, axis=1)
    tabtg = jnp.concatenate(
        [jnp.transpose(nx1_rows[:, :, :3], (0, 2, 1)), jnp.transpose(f1b, (0, 2, 1))],
        axis=1,
    )
    gg = _group(gq_rows, tabtg, 100.0, 64, mb=8)
    xg = gg[:, :, 0, :].reshape(_B * 64, 259)
    yg, stg = _mlp_chain(xg, params["gsa"])
    gf = _finalize_max(yg, stg, 64)

    point_coords = points[:, 0:4]
    return pf, point_coords, gf


# speedup vs baseline: 1.6743x; 1.0338x over previous
import functools

import jax
import jax.numpy as jnp
from jax.experimental import pallas as pl
from jax.experimental.pallas import tpu as pltpu

_B = 2
_NPER = 4096
_BIG = 1e9


def _fps_body(xyzt_ref, xyzr_ref, out_ref, *, npoint, n, b):
    xt = xyzt_ref[...]
    out_ref[:, 0:1, :] = xyzr_ref[:, 0:1, :]
    iota = jax.lax.broadcasted_iota(jnp.int32, (b, n), 1)
    col0 = jnp.sum(
        xt * (iota == 0).astype(jnp.float32)[:, None, :], axis=2, keepdims=True
    )

    def body(i, carry):
        dists, lastcol = carry
        d = jnp.sum((xt - lastcol) ** 2, axis=1)
        dists = jnp.minimum(dists, d)
        mx = jnp.max(dists, axis=1, keepdims=True)
        nxt = jnp.min(
            jnp.where(dists == mx, iota, jnp.int32(n)), axis=1, keepdims=True
        )
        oh = (iota == nxt).astype(jnp.float32)
        nxtcol = jnp.sum(xt * oh[:, None, :], axis=2, keepdims=True)
        out_ref[:, pl.ds(i, 1), :] = jnp.transpose(nxtcol, (0, 2, 1))
        return dists, nxtcol

    dists0 = jnp.full((b, n), 1e10, dtype=jnp.float32)
    jax.lax.fori_loop(1, npoint, body, (dists0, col0))


def _fps(xyz_rows, npoint):
    b, n, _ = xyz_rows.shape
    xyzt = jnp.transpose(xyz_rows, (0, 2, 1))
    return pl.pallas_call(
        functools.partial(_fps_body, npoint=npoint, n=n, b=b),
        grid=(1,),
        in_specs=[
            pl.BlockSpec((b, 8, n), lambda i: (0, 0, 0)),
            pl.BlockSpec((b, n, 8), lambda i: (0, 0, 0)),
        ],
        out_specs=pl.BlockSpec((b, npoint, 8), lambda i: (0, 0, 0)),
        out_shape=jax.ShapeDtypeStruct((b, npoint, 8), jnp.float32),
    )(xyzt, xyz_rows)


def _group_body(new_ref, tabt_ref, out_ref, *, r2, k, n, ct, mb):
    new = new_ref[0]
    tabt = tabt_ref[0]
    d2 = (new[:, 0:1] - tabt[0:1, :]) ** 2
    d2 = d2 + (new[:, 1:2] - tabt[1:2, :]) ** 2
    d2 = d2 + (new[:, 2:3] - tabt[2:3, :]) ** 2
    maskf = jnp.where(d2 < r2, 1.0, 0.0)
    lane = jax.lax.broadcasted_iota(jnp.int32, (mb, n), 1)
    rank = maskf
    sh = 1
    while sh < n:
        rolled = pltpu.roll(rank, sh, 1)
        rank = rank + jnp.where(lane >= sh, rolled, 0.0)
        sh *= 2
    count = rank[:, n - 1 : n]
    rc = rank * maskf
    offs = jnp.where(count > 0, new[:, 0:3], 0.0)
    offs_full = jnp.concatenate(
        [offs, jnp.zeros((mb, ct - 3), jnp.float32)], axis=1
    )
    g0 = None
    for s in range(k):
        oh = jnp.where(rc == float(s + 1), 1.0, 0.0)
        g = jax.lax.dot_general(
            oh, tabt, (((1,), (1,)), ((), ())),
            precision=jax.lax.Precision.HIGHEST,
            preferred_element_type=jnp.float32,
        )
        if s == 0:
            g0 = g
        else:
            g = g + jnp.where(count <= s, g0, 0.0)
        out_ref[0, s, :, :] = g - offs_full


def _group(new_rows, tabt, radius, k, mb):
    b, m, _ = new_rows.shape
    ct, n = tabt.shape[1], tabt.shape[2]
    return pl.pallas_call(
        functools.partial(
            _group_body, r2=radius * radius, k=k, n=n, ct=ct, mb=mb
        ),
        grid=(b, m // mb),
        in_specs=[
            pl.BlockSpec((1, mb, 8), lambda i, j: (i, j, 0)),
            pl.BlockSpec((1, ct, n), lambda i, j: (i, 0, 0)),
        ],
        out_specs=pl.BlockSpec((1, k, mb, ct), lambda i, j: (i, 0, j, 0)),
        out_shape=jax.ShapeDtypeStruct((b, k, m, ct), jnp.float32),
    )(new_rows, tabt)


def _layer_body(x_ref, st_ref, w_ref, y_ref, sum_ref, sq_ref, *, act):
    x = x_ref[...]
    a = x * st_ref[0:1, :] + st_ref[1:2, :]
    if act:
        a = jnp.maximum(a, 0.0)
    y = jax.lax.dot_general(
        a, w_ref[...], (((1,), (0,)), ((), ())), preferred_element_type=jnp.float32
    )
    y_ref[...] = y

    @pl.when(pl.program_id(0) == 0)
    def _():
        sum_ref[...] = jnp.zeros_like(sum_ref)
        sq_ref[...] = jnp.zeros_like(sq_ref)

    sum_ref[...] += jnp.sum(y, axis=0, keepdims=True)
    sq_ref[...] += jnp.sum(y * y, axis=0, keepdims=True)


def _layer(x, st, wt, act, rb):
    r, ci = x.shape
    co = wt.shape[1]
    rb = min(rb, r)
    return pl.pallas_call(
        functools.partial(_layer_body, act=act),
        grid=(r // rb,),
        in_specs=[
            pl.BlockSpec((rb, ci), lambda i: (i, 0)),
            pl.BlockSpec((8, ci), lambda i: (0, 0)),
            pl.BlockSpec((ci, co), lambda i: (0, 0)),
        ],
        out_specs=[
            pl.BlockSpec((rb, co), lambda i: (i, 0)),
            pl.BlockSpec((1, co), lambda i: (0, 0)),
            pl.BlockSpec((1, co), lambda i: (0, 0)),
        ],
        out_shape=[
            jax.ShapeDtypeStruct((r, co), jnp.float32),
            jax.ShapeDtypeStruct((1, co), jnp.float32),
            jax.ShapeDtypeStruct((1, co), jnp.float32),
        ],
    )(x, st, wt)


def _pack_st(s, t):
    st = jnp.zeros((8, s.shape[-1]), jnp.float32)
    return st.at[0].set(s).at[1].set(t)


def _bn_affine(ssum, ssq, ntot, g, b):
    mean = ssum[0] / ntot
    var = jnp.maximum(ssq[0] / ntot - mean * mean, 0.0)
    s = g * jax.lax.rsqrt(var + 1e-5)
    t = b - mean * s
    return _pack_st(s, t)


def _mlp_chain(x, layers, rb=2048):
    r = x.shape[0]
    st = _pack_st(jnp.ones((x.shape[1],), jnp.float32), jnp.zeros((x.shape[1],), jnp.float32))
    act = False
    y = x
    for lyr in layers:
        y, ssum, ssq = _layer(y, st, lyr["W"].T, act, rb)
        st = _bn_affine(ssum, ssq, float(r), lyr["g"], lyr["b"])
        act = True
    return y, st


def _finmax_body(y_ref, st_ref, o_ref, *, mb, k):
    a = jnp.maximum(y_ref[...] * st_ref[0:1, :] + st_ref[1:2, :], 0.0)
    c = a.shape[-1]
    o_ref[...] = jnp.max(a.reshape(mb, k, c), axis=1)


def _finalize_max(y, st, k):
    rk, c = y.shape
    rows = rk // k
    mb = min(rows, 256)
    return pl.pallas_call(
        functools.partial(_finmax_body, mb=mb, k=k),
        grid=(rows // mb,),
        in_specs=[
            pl.BlockSpec((mb * k, c), lambda i: (i, 0)),
            pl.BlockSpec((8, c), lambda i: (0, 0)),
        ],
        out_specs=pl.BlockSpec((mb, c), lambda i: (i, 0)),
        out_shape=jax.ShapeDtypeStruct((rows, c), jnp.float32),
    )(y, st)


def _finmax2_body(y_ref, st_ref, o_ref):
    a = jnp.maximum(y_ref[0] * st_ref[0:1, :] + st_ref[1:2, :], 0.0)
    o_ref[0] = jnp.max(a, axis=0)


def _finalize_max2(y4, st, mbf=128):
    b, k, m, c = y4.shape
    mbf = min(mbf, m)
    return pl.pallas_call(
        _finmax2_body,
        grid=(b, m // mbf),
        in_specs=[
            pl.BlockSpec((1, k, mbf, c), lambda i, j: (i, 0, j, 0)),
            pl.BlockSpec((8, c), lambda i, j: (0, 0)),
        ],
        out_specs=pl.BlockSpec((1, mbf, c), lambda i, j: (i, j, 0)),
        out_shape=jax.ShapeDtypeStruct((b, m, c), jnp.float32),
    )(y4, st)


def _fin_body(y_ref, st_ref, o_ref):
    o_ref[...] = jnp.maximum(y_ref[...] * st_ref[0:1, :] + st_ref[1:2, :], 0.0)


def _finalize(y, st, rb=2048):
    r, c = y.shape
    rb = min(rb, r)
    return pl.pallas_call(
        _fin_body,
        grid=(r // rb,),
        in_specs=[
            pl.BlockSpec((rb, c), lambda i: (i, 0)),
            pl.BlockSpec((8, c), lambda i: (0, 0)),
        ],
        out_specs=pl.BlockSpec((rb, c), lambda i: (i, 0)),
        out_shape=jax.ShapeDtypeStruct((r, c), jnp.float32),
    )(y, st)


def _sa_msg(new_rows, tabt, radii, nsamps, branches, mb):
    b, m, _ = new_rows.shape
    ct = tabt.shape[1]
    outs = []
    for radius, k, layers in zip(radii, nsamps, branches):
        g = _group(new_rows, tabt, radius, k, mb)
        x = g.reshape(b * k * m, ct)
        y, st = _mlp_chain(x, layers)
        c = y.shape[-1]
        outs.append(_finalize_max2(y.reshape(b, k, m, c), st))
    return jnp.concatenate(outs, axis=-1)


def _interp_body(unk_ref, kt_ref, kf_ref, skip_ref, out_ref, *, kn, c, ub):
    unk = unk_ref[0]
    kt = kt_ref[0]
    d2 = (unk[:, 0:1] - kt[0:1, :]) ** 2
    d2 = d2 + (unk[:, 1:2] - kt[1:2, :]) ** 2
    d2 = d2 + (unk[:, 2:3] - kt[2:3, :]) ** 2
    iota = jax.lax.broadcasted_iota(jnp.int32, (ub, kn), 1)
    wmat = jnp.zeros((ub, kn), jnp.float32)
    wsum = jnp.zeros((ub, 1), jnp.float32)
    cur = d2
    for _ in range(3):
        ms = jnp.min(cur, axis=1, keepdims=True)
        idx = jnp.min(
            jnp.where(cur == ms, iota, jnp.int32(kn)), axis=1, keepdims=True
        )
        oh = iota == idx
        w = 1.0 / (ms + 1e-8)
        wmat = wmat + jnp.where(oh, w, 0.0)
        wsum = wsum + w
        cur = jnp.where(oh, _BIG, cur)
    wmat = wmat / wsum
    interp = jax.lax.dot_general(
        wmat, kf_ref[0], (((1,), (0,)), ((), ())),
        precision=jax.lax.Precision.HIGHEST,
        preferred_element_type=jnp.float32,
    )
    out_ref[0, :, 0:c] = interp
    out_ref[0, :, c:] = skip_ref[0]


def _interp_concat(unk_rows, known_rows, kf, skip, ub):
    b, u, _ = unk_rows.shape
    kn = known_rows.shape[1]
    c = kf.shape[2]
    cu = skip.shape[2]
    kt = jnp.transpose(known_rows, (0, 2, 1))
    return pl.pallas_call(
        functools.partial(_interp_body, kn=kn, c=c, ub=ub),
        grid=(b, u // ub),
        in_specs=[
            pl.BlockSpec((1, ub, 8), lambda i, j: (i, j, 0)),
            pl.BlockSpec((1, 8, kn), lambda i, j: (i, 0, 0)),
            pl.BlockSpec((1, kn, c), lambda i, j: (i, 0, 0)),
            pl.BlockSpec((1, ub, cu), lambda i, j: (i, j, 0)),
        ],
        out_specs=pl.BlockSpec((1, ub, c + cu), lambda i, j: (i, j, 0)),
        out_shape=jax.ShapeDtypeStruct((b, u, c + cu), jnp.float32),
    )(unk_rows, kt, kf, skip)


def _fp(unk_rows, known_rows, kf, skip, layers, ub):
    x = _interp_concat(unk_rows, known_rows, kf, skip, ub)
    b, u, cx = x.shape
    y, st = _mlp_chain(x.reshape(b * u, cx), layers)
    return _finalize(y, st)


def kernel(points, params):
    points = jnp.asarray(points, jnp.float32)
    xyz = points[:, 1:4]
    xb = xyz.reshape(_B, _NPER, 3)
    xb_rows = jnp.pad(xb, ((0, 0), (0, 0), (0, 5)))
    feats = points[:, 4:].reshape(_B, _NPER, -1)

    nx0_rows = _fps(xb_rows, 1024)
    tabt0 = jnp.concatenate(
        [jnp.transpose(xb, (0, 2, 1)), jnp.transpose(feats, (0, 2, 1))], axis=1
    )
    f0b = _sa_msg(
        nx0_rows, tabt0, [0.4, 0.8], [16, 32], params["sa"][0], mb=64
    )

    nx1_rows = _fps(nx0_rows, 256)
    tabt1 = jnp.concatenate(
        [jnp.transpose(nx0_rows[:, :, :3], (0, 2, 1)), jnp.transpose(f0b, (0, 2, 1))],
        axis=1,
    )
    f1b = _sa_msg(
        nx1_rows, tabt1, [0.8, 1.6], [16, 32], params["sa"][1], mb=128
    )

    f0u = _fp(nx0_rows, nx1_rows, f1b, f0b, params["fp"][1], ub=256)
    raw = points[:, 1:].reshape(_B, _NPER, -1)
    pf = _fp(
        xb_rows, nx0_rows, f0u.reshape(_B, 1024, 128), raw, params["fp"][0], ub=256
    )

    gq_rows = jnp.repeat(nx1_rows[:, 0:1, :], 8, axis=1)
    tabtg = jnp.concatenate(
        [jnp.transpose(nx1_rows[:, :, :3], (0, 2, 1)), jnp.transpose(f1b, (0, 2, 1))],
        axis=1,
    )
    gg = _group(gq_rows, tabtg, 100.0, 64, mb=8)
    xg = gg[:, :, 0, :].reshape(_B * 64, 259)
    yg, stg = _mlp_chain(xg, params["gsa"])
    gf = _finalize_max(yg, stg, 64)

    point_coords = points[:, 0:4]
    return pf, point_coords, gf


# speedup vs baseline: 2.2581x; 1.3487x over previous
import functools

import jax
import jax.numpy as jnp
from jax.experimental import pallas as pl
from jax.experimental.pallas import tpu as pltpu

_B = 2
_NPER = 4096
_BIG = 1e9


def _fps_body(xyzt_ref, xyzr_ref, out_ref, *, npoint, n, b):
    xt = xyzt_ref[...]
    out_ref[:, 0:1, :] = xyzr_ref[:, 0:1, :]
    iota = jax.lax.broadcasted_iota(jnp.int32, (b, n), 1)
    col0 = jnp.sum(
        xt * (iota == 0).astype(jnp.float32)[:, None, :], axis=2, keepdims=True
    )

    def body(i, carry):
        dists, lastcol = carry
        d = jnp.sum((xt - lastcol) ** 2, axis=1)
        dists = jnp.minimum(dists, d)
        mx = jnp.max(dists, axis=1, keepdims=True)
        nxt = jnp.min(
            jnp.where(dists == mx, iota, jnp.int32(n)), axis=1, keepdims=True
        )
        oh = (iota == nxt).astype(jnp.float32)
        nxtcol = jnp.sum(xt * oh[:, None, :], axis=2, keepdims=True)
        out_ref[:, pl.ds(i, 1), :] = jnp.transpose(nxtcol, (0, 2, 1))
        return dists, nxtcol

    dists0 = jnp.full((b, n), 1e10, dtype=jnp.float32)
    jax.lax.fori_loop(1, npoint, body, (dists0, col0))


def _fps(xyz_rows, npoint):
    b, n, _ = xyz_rows.shape
    xyzt = jnp.transpose(xyz_rows, (0, 2, 1))
    return pl.pallas_call(
        functools.partial(_fps_body, npoint=npoint, n=n, b=b),
        grid=(1,),
        in_specs=[
            pl.BlockSpec((b, 8, n), lambda i: (0, 0, 0)),
            pl.BlockSpec((b, n, 8), lambda i: (0, 0, 0)),
        ],
        out_specs=pl.BlockSpec((b, npoint, 8), lambda i: (0, 0, 0)),
        out_shape=jax.ShapeDtypeStruct((b, npoint, 8), jnp.float32),
    )(xyzt, xyz_rows)


def _group_body(new_ref, tabt_ref, out_ref, *, r2, k, n, ct, mb):
    new = new_ref[0]
    tabt = tabt_ref[0]
    d2 = (new[:, 0:1] - tabt[0:1, :]) ** 2
    d2 = d2 + (new[:, 1:2] - tabt[1:2, :]) ** 2
    d2 = d2 + (new[:, 2:3] - tabt[2:3, :]) ** 2
    maskf = jnp.where(d2 < r2, 1.0, 0.0)
    lane = jax.lax.broadcasted_iota(jnp.int32, (mb, n), 1)
    rank = maskf
    sh = 1
    while sh < n:
        rolled = pltpu.roll(rank, sh, 1)
        rank = rank + jnp.where(lane >= sh, rolled, 0.0)
        sh *= 2
    count = rank[:, n - 1 : n]
    rc = jnp.minimum(rank * maskf, float(k + 1)).astype(jnp.bfloat16)
    offs = jnp.where(count > 0, new[:, 0:3], 0.0)
    offs_full = jnp.concatenate(
        [offs, jnp.zeros((mb, ct - 3), jnp.float32)], axis=1
    )
    t1 = tabt.astype(jnp.bfloat16)
    r1 = tabt - t1.astype(jnp.float32)
    t2 = r1.astype(jnp.bfloat16)
    t3 = (r1 - t2.astype(jnp.float32)).astype(jnp.bfloat16)
    g0 = None
    for s in range(k):
        oh = jnp.where(rc == jnp.bfloat16(s + 1), jnp.bfloat16(1), jnp.bfloat16(0))
        g = None
        for tp in (t1, t2, t3):
            gp = jax.lax.dot_general(
                oh, tp, (((1,), (1,)), ((), ())),
                preferred_element_type=jnp.float32,
            )
            g = gp if g is None else g + gp
        if s == 0:
            g0 = g
        else:
            g = g + jnp.where(count <= s, g0, 0.0)
        out_ref[0, s, :, :] = g - offs_full


def _group(new_rows, tabt, radius, k, mb):
    b, m, _ = new_rows.shape
    ct, n = tabt.shape[1], tabt.shape[2]
    return pl.pallas_call(
        functools.partial(
            _group_body, r2=radius * radius, k=k, n=n, ct=ct, mb=mb
        ),
        grid=(b, m // mb),
        in_specs=[
            pl.BlockSpec((1, mb, 8), lambda i, j: (i, j, 0)),
            pl.BlockSpec((1, ct, n), lambda i, j: (i, 0, 0)),
        ],
        out_specs=pl.BlockSpec((1, k, mb, ct), lambda i, j: (i, 0, j, 0)),
        out_shape=jax.ShapeDtypeStruct((b, k, m, ct), jnp.float32),
    )(new_rows, tabt)


def _layer_body(x_ref, st_ref, w_ref, y_ref, sum_ref, sq_ref, *, act):
    x = x_ref[...]
    a = x * st_ref[0:1, :] + st_ref[1:2, :]
    if act:
        a = jnp.maximum(a, 0.0)
    y = jax.lax.dot_general(
        a, w_ref[...], (((1,), (0,)), ((), ())), preferred_element_type=jnp.float32
    )
    y_ref[...] = y

    @pl.when(pl.program_id(0) == 0)
    def _():
        sum_ref[...] = jnp.zeros_like(sum_ref)
        sq_ref[...] = jnp.zeros_like(sq_ref)

    sum_ref[...] += jnp.sum(y, axis=0, keepdims=True)
    sq_ref[...] += jnp.sum(y * y, axis=0, keepdims=True)


def _layer(x, st, wt, act, rb):
    r, ci = x.shape
    co = wt.shape[1]
    rb = min(rb, r)
    return pl.pallas_call(
        functools.partial(_layer_body, act=act),
        grid=(r // rb,),
        in_specs=[
            pl.BlockSpec((rb, ci), lambda i: (i, 0)),
            pl.BlockSpec((8, ci), lambda i: (0, 0)),
            pl.BlockSpec((ci, co), lambda i: (0, 0)),
        ],
        out_specs=[
            pl.BlockSpec((rb, co), lambda i: (i, 0)),
            pl.BlockSpec((1, co), lambda i: (0, 0)),
            pl.BlockSpec((1, co), lambda i: (0, 0)),
        ],
        out_shape=[
            jax.ShapeDtypeStruct((r, co), jnp.float32),
            jax.ShapeDtypeStruct((1, co), jnp.float32),
            jax.ShapeDtypeStruct((1, co), jnp.float32),
        ],
    )(x, st, wt)


def _pack_st(s, t):
    st = jnp.zeros((8, s.shape[-1]), jnp.float32)
    return st.at[0].set(s).at[1].set(t)


def _bn_affine(ssum, ssq, ntot, g, b):
    mean = ssum[0] / ntot
    var = jnp.maximum(ssq[0] / ntot - mean * mean, 0.0)
    s = g * jax.lax.rsqrt(var + 1e-5)
    t = b - mean * s
    return _pack_st(s, t)


def _mlp_chain(x, layers, rb=2048):
    r = x.shape[0]
    st = _pack_st(jnp.ones((x.shape[1],), jnp.float32), jnp.zeros((x.shape[1],), jnp.float32))
    act = False
    y = x
    for lyr in layers:
        y, ssum, ssq = _layer(y, st, lyr["W"].T, act, rb)
        st = _bn_affine(ssum, ssq, float(r), lyr["g"], lyr["b"])
        act = True
    return y, st


def _finmax_body(y_ref, st_ref, o_ref, *, mb, k):
    a = jnp.maximum(y_ref[...] * st_ref[0:1, :] + st_ref[1:2, :], 0.0)
    c = a.shape[-1]
    o_ref[...] = jnp.max(a.reshape(mb, k, c), axis=1)


def _finalize_max(y, st, k):
    rk, c = y.shape
    rows = rk // k
    mb = min(rows, 256)
    return pl.pallas_call(
        functools.partial(_finmax_body, mb=mb, k=k),
        grid=(rows // mb,),
        in_specs=[
            pl.BlockSpec((mb * k, c), lambda i: (i, 0)),
            pl.BlockSpec((8, c), lambda i: (0, 0)),
        ],
        out_specs=pl.BlockSpec((mb, c), lambda i: (i, 0)),
        out_shape=jax.ShapeDtypeStruct((rows, c), jnp.float32),
    )(y, st)


def _finmax2_body(y_ref, st_ref, o_ref):
    a = jnp.maximum(y_ref[0] * st_ref[0:1, :] + st_ref[1:2, :], 0.0)
    o_ref[0] = jnp.max(a, axis=0)


def _finalize_max2(y4, st, mbf=128):
    b, k, m, c = y4.shape
    mbf = min(mbf, m)
    return pl.pallas_call(
        _finmax2_body,
        grid=(b, m // mbf),
        in_specs=[
            pl.BlockSpec((1, k, mbf, c), lambda i, j: (i, 0, j, 0)),
            pl.BlockSpec((8, c), lambda i, j: (0, 0)),
        ],
        out_specs=pl.BlockSpec((1, mbf, c), lambda i, j: (i, j, 0)),
        out_shape=jax.ShapeDtypeStruct((b, m, c), jnp.float32),
    )(y4, st)


def _fin_body(y_ref, st_ref, o_ref):
    o_ref[...] = jnp.maximum(y_ref[...] * st_ref[0:1, :] + st_ref[1:2, :], 0.0)


def _finalize(y, st, rb=2048):
    r, c = y.shape
    rb = min(rb, r)
    return pl.pallas_call(
        _fin_body,
        grid=(r // rb,),
        in_specs=[
            pl.BlockSpec((rb, c), lambda i: (i, 0)),
            pl.BlockSpec((8, c), lambda i: (0, 0)),
        ],
        out_specs=pl.BlockSpec((rb, c), lambda i: (i, 0)),
        out_shape=jax.ShapeDtypeStruct((r, c), jnp.float32),
    )(y, st)


def _sa_msg(new_rows, tabt, radii, nsamps, branches, mb):
    b, m, _ = new_rows.shape
    ct = tabt.shape[1]
    outs = []
    for radius, k, layers in zip(radii, nsamps, branches):
        g = _group(new_rows, tabt, radius, k, mb)
        x = g.reshape(b * k * m, ct)
        y, st = _mlp_chain(x, layers)
        c = y.shape[-1]
        outs.append(_finalize_max2(y.reshape(b, k, m, c), st))
    return jnp.concatenate(outs, axis=-1)


def _interp_body(unk_ref, kt_ref, kf_ref, skip_ref, out_ref, *, kn, c, ub):
    unk = unk_ref[0]
    kt = kt_ref[0]
    d2 = (unk[:, 0:1] - kt[0:1, :]) ** 2
    d2 = d2 + (unk[:, 1:2] - kt[1:2, :]) ** 2
    d2 = d2 + (unk[:, 2:3] - kt[2:3, :]) ** 2
    iota = jax.lax.broadcasted_iota(jnp.int32, (ub, kn), 1)
    wmat = jnp.zeros((ub, kn), jnp.float32)
    wsum = jnp.zeros((ub, 1), jnp.float32)
    cur = d2
    for _ in range(3):
        ms = jnp.min(cur, axis=1, keepdims=True)
        idx = jnp.min(
            jnp.where(cur == ms, iota, jnp.int32(kn)), axis=1, keepdims=True
        )
        oh = iota == idx
        w = 1.0 / (ms + 1e-8)
        wmat = wmat + jnp.where(oh, w, 0.0)
        wsum = wsum + w
        cur = jnp.where(oh, _BIG, cur)
    wmat = wmat / wsum
    interp = jax.lax.dot_general(
        wmat, kf_ref[0], (((1,), (0,)), ((), ())),
        precision=jax.lax.Precision.HIGHEST,
        preferred_element_type=jnp.float32,
    )
    out_ref[0, :, 0:c] = interp
    out_ref[0, :, c:] = skip_ref[0]


def _interp_concat(unk_rows, known_rows, kf, skip, ub):
    b, u, _ = unk_rows.shape
    kn = known_rows.shape[1]
    c = kf.shape[2]
    cu = skip.shape[2]
    kt = jnp.transpose(known_rows, (0, 2, 1))
    return pl.pallas_call(
        functools.partial(_interp_body, kn=kn, c=c, ub=ub),
        grid=(b, u // ub),
        in_specs=[
            pl.BlockSpec((1, ub, 8), lambda i, j: (i, j, 0)),
            pl.BlockSpec((1, 8, kn), lambda i, j: (i, 0, 0)),
            pl.BlockSpec((1, kn, c), lambda i, j: (i, 0, 0)),
            pl.BlockSpec((1, ub, cu), lambda i, j: (i, j, 0)),
        ],
        out_specs=pl.BlockSpec((1, ub, c + cu), lambda i, j: (i, j, 0)),
        out_shape=jax.ShapeDtypeStruct((b, u, c + cu), jnp.float32),
    )(unk_rows, kt, kf, skip)


def _fp(unk_rows, known_rows, kf, skip, layers, ub):
    x = _interp_concat(unk_rows, known_rows, kf, skip, ub)
    b, u, cx = x.shape
    y, st = _mlp_chain(x.reshape(b * u, cx), layers)
    return _finalize(y, st)


def kernel(points, params):
    points = jnp.asarray(points, jnp.float32)
    xyz = points[:, 1:4]
    xb = xyz.reshape(_B, _NPER, 3)
    xb_rows = jnp.pad(xb, ((0, 0), (0, 0), (0, 5)))
    feats = points[:, 4:].reshape(_B, _NPER, -1)

    nx0_rows = _fps(xb_rows, 1024)
    tabt0 = jnp.concatenate(
        [jnp.transpose(xb, (0, 2, 1)), jnp.transpose(feats, (0, 2, 1))], axis=1
    )
    f0b = _sa_msg(
        nx0_rows, tabt0, [0.4, 0.8], [16, 32], params["sa"][0], mb=64
    )

    nx1_rows = _fps(nx0_rows, 256)
    tabt1 = jnp.concatenate(
        [jnp.transpose(nx0_rows[:, :, :3], (0, 2, 1)), jnp.transpose(f0b, (0, 2, 1))],
        axis=1,
    )
    f1b = _sa_msg(
        nx1_rows, tabt1, [0.8, 1.6], [16, 32], params["sa"][1], mb=128
    )

    f0u = _fp(nx0_rows, nx1_rows, f1b, f0b, params["fp"][1], ub=256)
    raw = points[:, 1:].reshape(_B, _NPER, -1)
    pf = _fp(
        xb_rows, nx0_rows, f0u.reshape(_B, 1024, 128), raw, params["fp"][0], ub=256
    )

    gq_rows = jnp.repeat(nx1_rows[:, 0:1, :], 8, axis=1)
    tabtg = jnp.concatenate(
        [jnp.transpose(nx1_rows[:, :, :3], (0, 2, 1)), jnp.transpose(f1b, (0, 2, 1))],
        axis=1,
    )
    gg = _group(gq_rows, tabtg, 100.0, 64, mb=8)
    xg = gg[:, :, 0, :].reshape(_B * 64, 259)
    yg, stg = _mlp_chain(xg, params["gsa"])
    gf = _finalize_max(yg, stg, 64)

    point_coords = points[:, 0:4]
    return pf, point_coords, gf


# speedup vs baseline: 2.5420x; 1.1257x over previous
import functools

import jax
import jax.numpy as jnp
from jax.experimental import pallas as pl
from jax.experimental.pallas import tpu as pltpu

_B = 2
_NPER = 4096
_BIG = 1e9


def _fps_body(xyzt_ref, xyzr_ref, out_ref, *, npoint, n, b):
    xt = xyzt_ref[...]
    out_ref[:, 0:1, :] = xyzr_ref[:, 0:1, :]
    iota = jax.lax.broadcasted_iota(jnp.int32, (b, n), 1)
    col0 = jnp.sum(
        xt * (iota == 0).astype(jnp.float32)[:, None, :], axis=2, keepdims=True
    )

    def body(i, carry):
        dists, lastcol = carry
        d = jnp.sum((xt - lastcol) ** 2, axis=1)
        dists = jnp.minimum(dists, d)
        mx = jnp.max(dists, axis=1, keepdims=True)
        nxt = jnp.min(
            jnp.where(dists == mx, iota, jnp.int32(n)), axis=1, keepdims=True
        )
        oh = (iota == nxt).astype(jnp.float32)
        nxtcol = jnp.sum(xt * oh[:, None, :], axis=2, keepdims=True)
        out_ref[:, pl.ds(i, 1), :] = jnp.transpose(nxtcol, (0, 2, 1))
        return dists, nxtcol

    dists0 = jnp.full((b, n), 1e10, dtype=jnp.float32)
    jax.lax.fori_loop(1, npoint, body, (dists0, col0))


def _fps(xyz_rows, npoint):
    b, n, _ = xyz_rows.shape
    xyzt = jnp.transpose(xyz_rows, (0, 2, 1))
    return pl.pallas_call(
        functools.partial(_fps_body, npoint=npoint, n=n, b=b),
        grid=(1,),
        in_specs=[
            pl.BlockSpec((b, 8, n), lambda i: (0, 0, 0)),
            pl.BlockSpec((b, n, 8), lambda i: (0, 0, 0)),
        ],
        out_specs=pl.BlockSpec((b, npoint, 8), lambda i: (0, 0, 0)),
        out_shape=jax.ShapeDtypeStruct((b, npoint, 8), jnp.float32),
    )(xyzt, xyz_rows)


def _group_body(new_ref, tabt_ref, out_ref, *, r2, k, n, ct, mb):
    new = new_ref[0]
    tabt = tabt_ref[0]
    d2 = (new[:, 0:1] - tabt[0:1, :]) ** 2
    d2 = d2 + (new[:, 1:2] - tabt[1:2, :]) ** 2
    d2 = d2 + (new[:, 2:3] - tabt[2:3, :]) ** 2
    maskf = jnp.where(d2 < r2, 1.0, 0.0)
    lane = jax.lax.broadcasted_iota(jnp.int32, (mb, n), 1)
    rank = maskf
    sh = 1
    while sh < n:
        rolled = pltpu.roll(rank, sh, 1)
        rank = rank + jnp.where(lane >= sh, rolled, 0.0)
        sh *= 2
    count = rank[:, n - 1 : n]
    rc = jnp.minimum(rank * maskf, float(k + 1)).astype(jnp.bfloat16)
    offs = jnp.where(count > 0, new[:, 0:3], 0.0)
    offs_full = jnp.concatenate(
        [offs, jnp.zeros((mb, ct - 3), jnp.float32)], axis=1
    )
    t1 = tabt.astype(jnp.bfloat16)
    t2 = (tabt - t1.astype(jnp.float32)).astype(jnp.bfloat16)
    g0 = None
    for s in range(k):
        oh = jnp.where(rc == jnp.bfloat16(s + 1), jnp.bfloat16(1), jnp.bfloat16(0))
        g = None
        for tp in (t1, t2):
            gp = jax.lax.dot_general(
                oh, tp, (((1,), (1,)), ((), ())),
                preferred_element_type=jnp.float32,
            )
            g = gp if g is None else g + gp
        if s == 0:
            g0 = g
        else:
            g = g + jnp.where(count <= s, g0, 0.0)
        out_ref[0, s, :, :] = g - offs_full


def _group(new_rows, tabt, radius, k, mb):
    b, m, _ = new_rows.shape
    ct, n = tabt.shape[1], tabt.shape[2]
    return pl.pallas_call(
        functools.partial(
            _group_body, r2=radius * radius, k=k, n=n, ct=ct, mb=mb
        ),
        grid=(b, m // mb),
        in_specs=[
            pl.BlockSpec((1, mb, 8), lambda i, j: (i, j, 0)),
            pl.BlockSpec((1, ct, n), lambda i, j: (i, 0, 0)),
        ],
        out_specs=pl.BlockSpec((1, k, mb, ct), lambda i, j: (i, 0, j, 0)),
        out_shape=jax.ShapeDtypeStruct((b, k, m, ct), jnp.float32),
    )(new_rows, tabt)


def _layer_body(x_ref, st_ref, w_ref, y_ref, sum_ref, sq_ref, *, act):
    x = x_ref[...]
    a = x * st_ref[0:1, :] + st_ref[1:2, :]
    if act:
        a = jnp.maximum(a, 0.0)
    y = jax.lax.dot_general(
        a, w_ref[...], (((1,), (0,)), ((), ())), preferred_element_type=jnp.float32
    )
    y_ref[...] = y

    @pl.when(pl.program_id(0) == 0)
    def _():
        sum_ref[...] = jnp.zeros_like(sum_ref)
        sq_ref[...] = jnp.zeros_like(sq_ref)

    sum_ref[...] += jnp.sum(y, axis=0, keepdims=True)
    sq_ref[...] += jnp.sum(y * y, axis=0, keepdims=True)


def _layer(x, st, wt, act, rb):
    r, ci = x.shape
    co = wt.shape[1]
    rb = min(rb, r)
    return pl.pallas_call(
        functools.partial(_layer_body, act=act),
        grid=(r // rb,),
        in_specs=[
            pl.BlockSpec((rb, ci), lambda i: (i, 0)),
            pl.BlockSpec((8, ci), lambda i: (0, 0)),
            pl.BlockSpec((ci, co), lambda i: (0, 0)),
        ],
        out_specs=[
            pl.BlockSpec((rb, co), lambda i: (i, 0)),
            pl.BlockSpec((1, co), lambda i: (0, 0)),
            pl.BlockSpec((1, co), lambda i: (0, 0)),
        ],
        out_shape=[
            jax.ShapeDtypeStruct((r, co), jnp.float32),
            jax.ShapeDtypeStruct((1, co), jnp.float32),
            jax.ShapeDtypeStruct((1, co), jnp.float32),
        ],
    )(x, st, wt)


def _pack_st(s, t):
    st = jnp.zeros((8, s.shape[-1]), jnp.float32)
    return st.at[0].set(s).at[1].set(t)


def _bn_affine(ssum, ssq, ntot, g, b):
    mean = ssum[0] / ntot
    var = jnp.maximum(ssq[0] / ntot - mean * mean, 0.0)
    s = g * jax.lax.rsqrt(var + 1e-5)
    t = b - mean * s
    return _pack_st(s, t)


def _mlp_chain(x, layers, rb=2048):
    r = x.shape[0]
    st = _pack_st(jnp.ones((x.shape[1],), jnp.float32), jnp.zeros((x.shape[1],), jnp.float32))
    act = False
    y = x
    for lyr in layers:
        y, ssum, ssq = _layer(y, st, lyr["W"].T, act, rb)
        st = _bn_affine(ssum, ssq, float(r), lyr["g"], lyr["b"])
        act = True
    return y, st


def _finmax_body(y_ref, st_ref, o_ref, *, mb, k):
    a = jnp.maximum(y_ref[...] * st_ref[0:1, :] + st_ref[1:2, :], 0.0)
    c = a.shape[-1]
    o_ref[...] = jnp.max(a.reshape(mb, k, c), axis=1)


def _finalize_max(y, st, k):
    rk, c = y.shape
    rows = rk // k
    mb = min(rows, 256)
    return pl.pallas_call(
        functools.partial(_finmax_body, mb=mb, k=k),
        grid=(rows // mb,),
        in_specs=[
            pl.BlockSpec((mb * k, c), lambda i: (i, 0)),
            pl.BlockSpec((8, c), lambda i: (0, 0)),
        ],
        out_specs=pl.BlockSpec((mb, c), lambda i: (i, 0)),
        out_shape=jax.ShapeDtypeStruct((rows, c), jnp.float32),
    )(y, st)


def _finmax2_body(y_ref, st_ref, o_ref):
    a = jnp.maximum(y_ref[0] * st_ref[0:1, :] + st_ref[1:2, :], 0.0)
    o_ref[0] = jnp.max(a, axis=0)


def _finalize_max2(y4, st, mbf=128):
    b, k, m, c = y4.shape
    mbf = min(mbf, m)
    return pl.pallas_call(
        _finmax2_body,
        grid=(b, m // mbf),
        in_specs=[
            pl.BlockSpec((1, k, mbf, c), lambda i, j: (i, 0, j, 0)),
            pl.BlockSpec((8, c), lambda i, j: (0, 0)),
        ],
        out_specs=pl.BlockSpec((1, mbf, c), lambda i, j: (i, j, 0)),
        out_shape=jax.ShapeDtypeStruct((b, m, c), jnp.float32),
    )(y4, st)


def _fin_body(y_ref, st_ref, o_ref):
    o_ref[...] = jnp.maximum(y_ref[...] * st_ref[0:1, :] + st_ref[1:2, :], 0.0)


def _finalize(y, st, rb=2048):
    r, c = y.shape
    rb = min(rb, r)
    return pl.pallas_call(
        _fin_body,
        grid=(r // rb,),
        in_specs=[
            pl.BlockSpec((rb, c), lambda i: (i, 0)),
            pl.BlockSpec((8, c), lambda i: (0, 0)),
        ],
        out_specs=pl.BlockSpec((rb, c), lambda i: (i, 0)),
        out_shape=jax.ShapeDtypeStruct((r, c), jnp.float32),
    )(y, st)


def _sa_msg(new_rows, tabt, radii, nsamps, branches, mb):
    b, m, _ = new_rows.shape
    ct = tabt.shape[1]
    outs = []
    for radius, k, layers in zip(radii, nsamps, branches):
        g = _group(new_rows, tabt, radius, k, mb)
        x = g.reshape(b * k * m, ct)
        y, st = _mlp_chain(x, layers)
        c = y.shape[-1]
        outs.append(_finalize_max2(y.reshape(b, k, m, c), st))
    return jnp.concatenate(outs, axis=-1)


def _interp_body(unk_ref, kt_ref, kf_ref, skip_ref, out_ref, *, kn, c, ub):
    unk = unk_ref[0]
    kt = kt_ref[0]
    d2 = (unk[:, 0:1] - kt[0:1, :]) ** 2
    d2 = d2 + (unk[:, 1:2] - kt[1:2, :]) ** 2
    d2 = d2 + (unk[:, 2:3] - kt[2:3, :]) ** 2
    iota = jax.lax.broadcasted_iota(jnp.int32, (ub, kn), 1)
    wmat = jnp.zeros((ub, kn), jnp.float32)
    wsum = jnp.zeros((ub, 1), jnp.float32)
    cur = d2
    for _ in range(3):
        ms = jnp.min(cur, axis=1, keepdims=True)
        idx = jnp.min(
            jnp.where(cur == ms, iota, jnp.int32(kn)), axis=1, keepdims=True
        )
        oh = iota == idx
        w = 1.0 / (ms + 1e-8)
        wmat = wmat + jnp.where(oh, w, 0.0)
        wsum = wsum + w
        cur = jnp.where(oh, _BIG, cur)
    wmat = wmat / wsum
    interp = jax.lax.dot_general(
        wmat, kf_ref[0], (((1,), (0,)), ((), ())),
        precision=jax.lax.Precision.HIGHEST,
        preferred_element_type=jnp.float32,
    )
    out_ref[0, :, 0:c] = interp
    out_ref[0, :, c:] = skip_ref[0]


def _interp_concat(unk_rows, known_rows, kf, skip, ub):
    b, u, _ = unk_rows.shape
    kn = known_rows.shape[1]
    c = kf.shape[2]
    cu = skip.shape[2]
    kt = jnp.transpose(known_rows, (0, 2, 1))
    return pl.pallas_call(
        functools.partial(_interp_body, kn=kn, c=c, ub=ub),
        grid=(b, u // ub),
        in_specs=[
            pl.BlockSpec((1, ub, 8), lambda i, j: (i, j, 0)),
            pl.BlockSpec((1, 8, kn), lambda i, j: (i, 0, 0)),
            pl.BlockSpec((1, kn, c), lambda i, j: (i, 0, 0)),
            pl.BlockSpec((1, ub, cu), lambda i, j: (i, j, 0)),
        ],
        out_specs=pl.BlockSpec((1, ub, c + cu), lambda i, j: (i, j, 0)),
        out_shape=jax.ShapeDtypeStruct((b, u, c + cu), jnp.float32),
    )(unk_rows, kt, kf, skip)


def _fp(unk_rows, known_rows, kf, skip, layers, ub):
    x = _interp_concat(unk_rows, known_rows, kf, skip, ub)
    b, u, cx = x.shape
    y, st = _mlp_chain(x.reshape(b * u, cx), layers)
    return _finalize(y, st)


def kernel(points, params):
    points = jnp.asarray(points, jnp.float32)
    xyz = points[:, 1:4]
    xb = xyz.reshape(_B, _NPER, 3)
    xb_rows = jnp.pad(xb, ((0, 0), (0, 0), (0, 5)))
    feats = points[:, 4:].reshape(_B, _NPER, -1)

    nx0_rows = _fps(xb_rows, 1024)
    tabt0 = jnp.concatenate(
        [jnp.transpose(xb, (0, 2, 1)), jnp.transpose(feats, (0, 2, 1))], axis=1
    )
    f0b = _sa_msg(
        nx0_rows, tabt0, [0.4, 0.8], [16, 32], params["sa"][0], mb=64
    )

    nx1_rows = _fps(nx0_rows, 256)
    tabt1 = jnp.concatenate(
        [jnp.transpose(nx0_rows[:, :, :3], (0, 2, 1)), jnp.transpose(f0b, (0, 2, 1))],
        axis=1,
    )
    f1b = _sa_msg(
        nx1_rows, tabt1, [0.8, 1.6], [16, 32], params["sa"][1], mb=128
    )

    f0u = _fp(nx0_rows, nx1_rows, f1b, f0b, params["fp"][1], ub=256)
    raw = points[:, 1:].reshape(_B, _NPER, -1)
    pf = _fp(
        xb_rows, nx0_rows, f0u.reshape(_B, 1024, 128), raw, params["fp"][0], ub=256
    )

    gq_rows = jnp.repeat(nx1_rows[:, 0:1, :], 8, axis=1)
    tabtg = jnp.concatenate(
        [jnp.transpose(nx1_rows[:, :, :3], (0, 2, 1)), jnp.transpose(f1b, (0, 2, 1))],
        axis=1,
    )
    gg = _group(gq_rows, tabtg, 100.0, 64, mb=8)
    xg = gg[:, :, 0, :].reshape(_B * 64, 259)
    yg, stg = _mlp_chain(xg, params["gsa"])
    gf = _finalize_max(yg, stg, 64)

    point_coords = points[:, 0:4]
    return pf, point_coords, gf
